# baseline (speedup 1.0000x reference)
import functools
import math

import jax
import jax.numpy as jnp
from jax import lax
from jax.experimental import pallas as pl
from jax.experimental.pallas import tpu as pltpu
from jax.experimental.pallas import tpu_sc as plsc

D_MODEL = 64
_SCALE = math.sqrt(D_MODEL)


@functools.lru_cache(maxsize=None)
def _build(V, D, B):
    info = plsc.get_sparse_core_info()
    NC, NS, L = info.num_cores, info.num_subcores, info.num_lanes
    NW = NC * NS
    assert B % NW == 0
    b_per_w = B // NW
    G = 128
    C = 256
    NG = C // G
    assert b_per_w % C == 0
    n_chunks = b_per_w // C
    NBUF = 2
    mesh = plsc.VectorSubcoreMesh(core_axis_name="c", subcore_axis_name="s")

    @functools.partial(
        pl.kernel,
        mesh=mesh,
        out_type=jax.ShapeDtypeStruct((B, D), jnp.float32),
        compiler_params=pltpu.CompilerParams(use_tc_tiling_on_sc=False),
        scratch_types=[
            pltpu.VMEM((n_chunks, C), jnp.int32),
            pltpu.VMEM((NBUF, C, D), jnp.float32),
            pltpu.VMEM((NBUF, C, D), jnp.float32),
            [pltpu.SemaphoreType.DMA] * NBUF,
            [pltpu.SemaphoreType.DMA] * NBUF,
        ],
    )
    def emb_kernel(table_hbm, x_hbm, out_hbm, idx_v, gbuf, wbuf, gsems, wsems):
        wid = lax.axis_index("s") * NC + lax.axis_index("c")
        base = wid * b_per_w
        pltpu.sync_copy(x_hbm.at[wid], idx_v)

        def start_gather(ci, b):
            for g in range(NG):
                pltpu.async_copy(
                    table_hbm.at[idx_v.at[ci, pl.ds(g * G, G)]],
                    gbuf.at[b, pl.ds(g * G, G)],
                    gsems[b],
                )

        def wait_gather(ci, b):
            for g in range(NG):
                pltpu.make_async_copy(
                    table_hbm.at[idx_v.at[ci, pl.ds(g * G, G)]],
                    gbuf.at[b, pl.ds(g * G, G)],
                    gsems[b],
                ).wait()

        def wait_write(b):
            pltpu.make_async_copy(
                wbuf.at[b], out_hbm.at[pl.ds(base, C)], wsems[b]
            ).wait()

        def start_write(ci, b):
            pltpu.async_copy(
                wbuf.at[b], out_hbm.at[pl.ds(base + ci * C, C)], wsems[b]
            )

        def scale(b):
            @plsc.parallel_loop(0, C, unroll=8)
            def _scale_body(r):
                for d in range(D // L):
                    sl = pl.ds(d * L, L)
                    wbuf[b, r, sl] = gbuf[b, r, sl] * _SCALE

        for b in range(NBUF):
            start_gather(b, b)

        for b in range(NBUF):
            wait_gather(b, b)
            scale(b)
            start_gather(b + NBUF, b)
            start_write(b, b)

        def steady(g0, carry):
            for b in range(NBUF):
                ci = g0 + b
                wait_gather(ci, b)
                wait_write(b)
                scale(b)
                start_gather(ci + NBUF, b)
                start_write(ci, b)
            return carry

        lax.fori_loop(1, n_chunks // NBUF - 1, lambda g, c: steady(g * NBUF, c), 0)

        for b in range(NBUF):
            ci = n_chunks - NBUF + b
            wait_gather(ci, b)
            wait_write(b)
            scale(b)
            start_write(ci, b)
        for b in range(NBUF):
            wait_write(b)

    def run(table, x_flat):
        x3 = x_flat.reshape(NW, n_chunks, C)
        return emb_kernel(table, x3)

    return run


def kernel(x, table):
    Bdim, T = x.shape
    V, D = table.shape
    run = _build(V, D, Bdim * T)
    out = run(table, x.reshape(-1).astype(jnp.int32))
    return out.reshape(Bdim, T, D)


# speedup vs baseline: 1.2239x; 1.2239x over previous
import functools
import math

import jax
import jax.numpy as jnp
from jax import lax
from jax.experimental import pallas as pl
from jax.experimental.pallas import tpu as pltpu
from jax.experimental.pallas import tpu_sc as plsc

D_MODEL = 64
_SCALE = math.sqrt(D_MODEL)
_LANES = 128


@functools.lru_cache(maxsize=None)
def _build(V, D, B):
    info = plsc.get_sparse_core_info()
    NC, NS, L = info.num_cores, info.num_subcores, info.num_lanes
    NW = NC * NS
    assert B % NW == 0
    b_per_w = B // NW
    C = 128
    assert b_per_w % C == 0
    n_chunks = b_per_w // C
    NBUF = 2
    mesh = plsc.VectorSubcoreMesh(core_axis_name="c", subcore_axis_name="s")

    @functools.partial(
        pl.kernel,
        mesh=mesh,
        out_type=jax.ShapeDtypeStruct((B, D), jnp.float32),
        compiler_params=pltpu.CompilerParams(use_tc_tiling_on_sc=True),
        scratch_types=[
            pltpu.VMEM((n_chunks, C), jnp.int32),
            pltpu.VMEM((NBUF, C, _LANES), jnp.float32),
            pltpu.VMEM((NBUF, C, D), jnp.float32),
            [pltpu.SemaphoreType.DMA] * NBUF,
            [pltpu.SemaphoreType.DMA] * NBUF,
        ],
    )
    def emb_kernel(table_hbm, x_hbm, out_hbm, idx_v, gbuf, wbuf, gsems, wsems):
        wid = lax.axis_index("s") * NC + lax.axis_index("c")
        base = wid * b_per_w
        pltpu.sync_copy(x_hbm.at[wid], idx_v)

        def start_gather(ci, b):
            pltpu.async_copy(table_hbm.at[idx_v.at[ci]], gbuf.at[b], gsems[b])

        def wait_gather(ci, b):
            pltpu.make_async_copy(
                table_hbm.at[idx_v.at[ci]], gbuf.at[b], gsems[b]
            ).wait()

        def wait_write(b):
            pltpu.make_async_copy(
                wbuf.at[b], out_hbm.at[pl.ds(base, C)], wsems[b]
            ).wait()

        def start_write(ci, b):
            pltpu.async_copy(
                wbuf.at[b], out_hbm.at[pl.ds(base + ci * C, C)], wsems[b]
            )

        def scale(b):
            @plsc.parallel_loop(0, C, unroll=8)
            def _scale_body(r):
                for d in range(D // L):
                    sl = pl.ds(d * L, L)
                    wbuf[b, r, sl] = gbuf[b, r, sl] * _SCALE

        for b in range(NBUF):
            start_gather(b, b)

        for b in range(NBUF):
            wait_gather(b, b)
            scale(b)
            start_gather(b + NBUF, b)
            start_write(b, b)

        def steady(g0, carry):
            for b in range(NBUF):
                ci = g0 + b
                wait_gather(ci, b)
                wait_write(b)
                scale(b)
                start_gather(ci + NBUF, b)
                start_write(ci, b)
            return carry

        lax.fori_loop(1, n_chunks // NBUF - 1, lambda g, c: steady(g * NBUF, c), 0)

        for b in range(NBUF):
            ci = n_chunks - NBUF + b
            wait_gather(ci, b)
            wait_write(b)
            scale(b)
            start_write(ci, b)
        for b in range(NBUF):
            wait_write(b)

    def run(table, x):
        table_p = jnp.pad(table, ((0, 0), (0, _LANES - D)))
        x3 = x.reshape(NW, n_chunks, C)
        return emb_kernel(table_p, x3)

    return run


def kernel(x, table):
    Bdim, T = x.shape
    V, D = table.shape
    run = _build(V, D, Bdim * T)
    out = run(table, x.reshape(-1).astype(jnp.int32))
    return out.reshape(Bdim, T, D)
